# Initial kernel scaffold; baseline (speedup 1.0000x reference)
import jax
import jax.numpy as jnp
from jax.experimental import pallas as pl


def kernel(x, edge_index_rel0, edge_index_rel1, embed_w1, embed_b1, embed_w2, embed_b2, gat_w, gat_al, gat_ar, gat_b, dec_w1, dec_b1, dec_w2, dec_b2):
    raise NotImplementedError("write your pallas kernel here")


# speedup vs baseline: 25.8355x; 25.8355x over previous
import functools

import jax
import jax.numpy as jnp
from jax import lax
from jax.experimental import pallas as pl
from jax.experimental.pallas import tpu as pltpu
from jax.experimental.pallas import tpu_sc as plsc

F32 = jnp.float32
NW = 32
KW = 128
BN = 1000


def _mesh():
    return plsc.VectorSubcoreMesh(core_axis_name="c", subcore_axis_name="s")


def _make_pass_a(n, npad, epad):
    ept = epad // NW
    nwin = ept // KW
    rpt = npad // 16

    def body(el_hbm, er_hbm, src_hbm, dst_hbm, z16_hbm,
             ee_hbm, den_hbm,
             src_v, dst_v, elr_v, err_v, ee_v, den_s):
        cid = lax.axis_index("c")
        sid = lax.axis_index("s")
        wid = sid * 2 + cid
        pltpu.sync_copy(z16_hbm, den_s.at[pl.ds(sid * rpt, rpt)])
        plsc.subcore_barrier()

        def win(w, carry):
            base = wid * ept + w * KW
            pltpu.sync_copy(src_hbm.at[pl.ds(base, KW)], src_v)
            pltpu.sync_copy(dst_hbm.at[pl.ds(base, KW)], dst_v)
            pltpu.sync_copy(el_hbm.at[src_v], elr_v)
            pltpu.sync_copy(er_hbm.at[dst_v], err_v)

            def ed(k, c2):
                s = elr_v[k, :] + err_v[k, :]
                s = jnp.maximum(s, s * 0.2)
                ee_v[k, :] = jnp.exp(s)
                return c2

            lax.fori_loop(0, KW, ed, 0)
            pltpu.sync_copy(ee_v, ee_hbm.at[pl.ds(base, KW)])
            pltpu.sync_copy(ee_v, den_s.at[dst_v], add=True)
            return carry

        lax.fori_loop(0, nwin, win, 0)
        plsc.subcore_barrier()
        pltpu.sync_copy(den_s.at[pl.ds(sid * rpt, rpt)],
                        den_hbm.at[cid, pl.ds(sid * rpt, rpt)])

    return pl.kernel(
        body,
        out_type=[jax.ShapeDtypeStruct((epad, 16), F32),
                  jax.ShapeDtypeStruct((2, npad, 16), F32)],
        mesh=_mesh(),
        compiler_params=pltpu.CompilerParams(use_tc_tiling_on_sc=False, needs_layout_passes=False),
        scratch_types=[
            pltpu.VMEM((KW,), jnp.int32),
            pltpu.VMEM((KW,), jnp.int32),
            pltpu.VMEM((KW, 16), F32),
            pltpu.VMEM((KW, 16), F32),
            pltpu.VMEM((KW, 16), F32),
            pltpu.VMEM_SHARED((npad, 16), F32),
        ],
    )


def _make_pass_b(n, npad, epad):
    ept = epad // NW
    nwin = ept // KW
    rpt = npad // 16

    def body(h0_hbm, h1_hbm, h2_hbm, h3_hbm, ee_hbm, src_hbm, dst_hbm,
             z32_hbm,
             r0_hbm, r1_hbm, r2_hbm, r3_hbm,
             src_v, dst_v, ee_v, h_v, m_v, acc_s):
        cid = lax.axis_index("c")
        sid = lax.axis_index("s")
        wid = sid * 2 + cid
        iota = lax.iota(jnp.int32, 16)
        hc_hbm = (h0_hbm, h1_hbm, h2_hbm, h3_hbm)
        raw_hbm = (r0_hbm, r1_hbm, r2_hbm, r3_hbm)

        for c in range(4):
            pat0 = 4 * c + lax.shift_right_logical(iota, 3)
            pat1 = 4 * c + 2 + lax.shift_right_logical(iota, 3)
            pltpu.sync_copy(z32_hbm, acc_s.at[pl.ds(sid * rpt, rpt)])
            plsc.subcore_barrier()

            def win(w, carry, _c=c, _p0=pat0, _p1=pat1):
                base = wid * ept + w * KW
                pltpu.sync_copy(src_hbm.at[pl.ds(base, KW)], src_v)
                pltpu.sync_copy(dst_hbm.at[pl.ds(base, KW)], dst_v)
                pltpu.sync_copy(ee_hbm.at[pl.ds(base, KW)], ee_v)
                pltpu.sync_copy(hc_hbm[_c].at[src_v], h_v)

                def ed(k, c2):
                    rows = jnp.full((16,), k, jnp.int32)
                    e0 = plsc.load_gather(ee_v, [rows, _p0])
                    e1 = plsc.load_gather(ee_v, [rows, _p1])
                    m_v[k, pl.ds(0, 16)] = h_v[k, pl.ds(0, 16)] * e0
                    m_v[k, pl.ds(16, 16)] = h_v[k, pl.ds(16, 16)] * e1
                    return c2

                lax.fori_loop(0, KW, ed, 0)
                pltpu.sync_copy(m_v, acc_s.at[dst_v], add=True)
                return carry

            lax.fori_loop(0, nwin, win, 0)
            plsc.subcore_barrier()
            pltpu.sync_copy(acc_s.at[pl.ds(sid * rpt, rpt)],
                            raw_hbm[c].at[cid, pl.ds(sid * rpt, rpt)])
            plsc.subcore_barrier()

    out = jax.ShapeDtypeStruct((2, npad, 32), F32)
    return pl.kernel(
        body,
        out_type=[out, out, out, out],
        mesh=_mesh(),
        compiler_params=pltpu.CompilerParams(use_tc_tiling_on_sc=False, needs_layout_passes=False),
        scratch_types=[
            pltpu.VMEM((KW,), jnp.int32),
            pltpu.VMEM((KW,), jnp.int32),
            pltpu.VMEM((KW, 16), F32),
            pltpu.VMEM((KW, 32), F32),
            pltpu.VMEM((KW, 32), F32),
            pltpu.VMEM_SHARED((npad, 32), F32),
        ],
    )


def _mlp2(x, w1, b1, w2, b2, relu_mid=True):
    n = x.shape[0]
    dout = w2.shape[1]

    def body(x_ref, w1_ref, b1_ref, w2_ref, b2_ref, o_ref):
        h = jnp.dot(x_ref[...], w1_ref[...], preferred_element_type=F32)
        h = jnp.maximum(h + b1_ref[...], 0.0)
        o_ref[...] = jnp.dot(h, w2_ref[...], preferred_element_type=F32) + b2_ref[...]

    return pl.pallas_call(
        body,
        grid=(n // BN,),
        in_specs=[pl.BlockSpec((BN, 128), lambda i: (i, 0)),
                  pl.BlockSpec((128, 128), lambda i: (0, 0)),
                  pl.BlockSpec((1, 128), lambda i: (0, 0)),
                  pl.BlockSpec((128, dout), lambda i: (0, 0)),
                  pl.BlockSpec((1, dout), lambda i: (0, 0))],
        out_specs=pl.BlockSpec((BN, dout), lambda i: (i, 0)),
        out_shape=jax.ShapeDtypeStruct((n, dout), F32),
    )(x, w1, b1.reshape(1, -1), w2, b2.reshape(1, -1))


def _project(h1, w, aal, aar):
    n = h1.shape[0]

    def body(h1_ref, w_ref, al_ref, ar_ref, *outs):
        h1b = h1_ref[...]
        for r in range(2):
            h = jnp.dot(h1b, w_ref[r], preferred_element_type=F32)
            for c in range(4):
                outs[r * 4 + c][...] = h[:, c * 32:(c + 1) * 32]
            outs[8 + 2 * r][...] = jnp.dot(h, al_ref[r], preferred_element_type=F32)
            outs[8 + 2 * r + 1][...] = jnp.dot(h, ar_ref[r], preferred_element_type=F32)

    chunk = jax.ShapeDtypeStruct((n, 32), F32)
    att = jax.ShapeDtypeStruct((n, 16), F32)
    return pl.pallas_call(
        body,
        grid=(n // BN,),
        in_specs=[pl.BlockSpec((BN, 128), lambda i: (i, 0)),
                  pl.BlockSpec((2, 128, 128), lambda i: (0, 0, 0)),
                  pl.BlockSpec((2, 128, 16), lambda i: (0, 0, 0)),
                  pl.BlockSpec((2, 128, 16), lambda i: (0, 0, 0))],
        out_specs=[pl.BlockSpec((BN, 32), lambda i: (i, 0))] * 8
        + [pl.BlockSpec((BN, 16), lambda i: (i, 0))] * 4,
        out_shape=[chunk] * 8 + [att] * 4,
    )(h1, w, aal, aar)


def _combine(raws0, raws1, den0, den1, h1, b, npad):
    n = h1.shape[0]

    def body(r00, r01, r02, r03, r10, r11, r12, r13, d0_ref, d1_ref,
             h1_ref, b_ref, o_ref):
        raws = ((r00, r01, r02, r03), (r10, r11, r12, r13))
        dens = (d0_ref, d1_ref)
        reps = []
        for r in range(2):
            d = dens[r][0] + dens[r][1]
            d = jnp.where(d == 0.0, 1.0, d)
            reps.append(jnp.repeat(1.0 / d, 8, axis=1))
        cols = []
        for c in range(4):
            col = jnp.zeros((BN, 32), F32)
            for r in range(2):
                s = raws[r][c][0] + raws[r][c][1]
                col = col + s * reps[r][:, c * 32:(c + 1) * 32] \
                    + b_ref[r, c * 32:(c + 1) * 32]
            cols.append(col)
        acc = jnp.concatenate(cols, axis=1)
        o_ref[...] = jnp.where(acc > 0, acc, 0.01 * acc) + h1_ref[...]

    rspec = pl.BlockSpec((2, BN, 32), lambda i: (0, i, 0))
    dspec = pl.BlockSpec((2, BN, 16), lambda i: (0, i, 0))
    return pl.pallas_call(
        body,
        grid=(n // BN,),
        in_specs=[rspec] * 8 + [dspec] * 2
        + [pl.BlockSpec((BN, 128), lambda i: (i, 0)),
           pl.BlockSpec((2, 128), lambda i: (0, 0))],
        out_specs=pl.BlockSpec((BN, 128), lambda i: (i, 0)),
        out_shape=jax.ShapeDtypeStruct((n, 128), F32),
    )(*raws0, *raws1, den0, den1, h1, b)


def kernel(x, edge_index_rel0, edge_index_rel1,
           embed_w1, embed_b1, embed_w2, embed_b2,
           gat_w, gat_al, gat_ar, gat_b,
           dec_w1, dec_b1, dec_w2, dec_b2):
    n = x.shape[0]
    e = edge_index_rel0.shape[1]
    L, R, heads, hd = gat_al.shape
    npad = ((n + 64 + 127) // 128) * 128
    epad = ((e + NW * KW - 1) // (NW * KW)) * (NW * KW)
    pad = epad - e
    rpt = npad // 16

    padsrc = jnp.zeros((pad,), jnp.int32)
    paddst = n + (jnp.arange(pad, dtype=jnp.int32) % 64)
    edges = []
    for ei in (edge_index_rel0, edge_index_rel1):
        edges.append((jnp.concatenate([ei[0], padsrc]),
                      jnp.concatenate([ei[1], paddst])))

    eye = (jnp.arange(128)[:, None] // hd == jnp.arange(heads)[None, :])
    aal = jnp.where(eye[None, None], gat_al.reshape(L, R, 128)[..., None], 0.0)
    aar = jnp.where(eye[None, None], gat_ar.reshape(L, R, 128)[..., None], 0.0)

    z16 = jnp.zeros((rpt, 16), F32)
    z32 = jnp.zeros((rpt, 32), F32)
    zpad16 = jnp.zeros((64, 16), F32)

    pass_a = _make_pass_a(n, npad, epad)
    pass_b = _make_pass_b(n, npad, epad)

    h1 = _mlp2(x, embed_w1, embed_b1, embed_w2, embed_b2)
    for l in range(L):
        proj = _project(h1, gat_w[l], aal[l], aar[l])
        hcs = (proj[0:4], proj[4:8])
        els = (proj[8], proj[10])
        ers = (proj[9], proj[11])
        raws, dens = [], []
        for r in range(R):
            erp = jnp.concatenate([ers[r], zpad16], axis=0)
            ee, den = pass_a(els[r], erp, edges[r][0], edges[r][1], z16)
            raw = pass_b(*hcs[r], ee, edges[r][0], edges[r][1], z32)
            raws.append(raw)
            dens.append(den)
        h1 = _combine(raws[0], raws[1], dens[0], dens[1], h1, gat_b[l], npad)
    return _mlp2(h1, dec_w1, dec_b1, dec_w2, dec_b2)


# speedup vs baseline: 34.9980x; 1.3546x over previous
import functools

import jax
import jax.numpy as jnp
from jax import lax
from jax.experimental import pallas as pl
from jax.experimental.pallas import tpu as pltpu
from jax.experimental.pallas import tpu_sc as plsc

F32 = jnp.float32
NW = 32
KW = 128
BN = 1000


def _mesh():
    return plsc.VectorSubcoreMesh(core_axis_name="c", subcore_axis_name="s")


def _make_pass_a(n, npad, epad):
    ept = epad // NW
    nwin = ept // KW
    rpt = npad // 16

    def body(el_hbm, er_hbm, src_hbm, dst_hbm, z16_hbm,
             ee_hbm, den_hbm,
             s0, s1, d0, d1, elr0, elr1, err0, err1, ee_v, den_s,
             rs0, rs1):
        cid = lax.axis_index("c")
        sid = lax.axis_index("s")
        wid = sid * 2 + cid
        tb = wid * ept
        sv, dv = (s0, s1), (d0, d1)
        elr, err, rsem = (elr0, elr1), (err0, err1), (rs0, rs1)
        pltpu.sync_copy(z16_hbm, den_s.at[pl.ds(sid * rpt, rpt)])
        plsc.subcore_barrier()

        def rows(w, p):
            return (pltpu.make_async_copy(el_hbm.at[sv[p]], elr[p], rsem[p]),
                    pltpu.make_async_copy(er_hbm.at[dv[p]], err[p], rsem[p]))

        pltpu.sync_copy(src_hbm.at[pl.ds(tb, KW)], s0)
        pltpu.sync_copy(dst_hbm.at[pl.ds(tb, KW)], d0)
        for cpy in rows(0, 0):
            cpy.start()

        def pair(j, carry):
            for p in range(2):
                w = 2 * j + p
                p1 = (p + 1) % 2
                nb = tb + lax.rem(w + 1, nwin) * KW
                pltpu.sync_copy(src_hbm.at[pl.ds(nb, KW)], sv[p1])
                pltpu.sync_copy(dst_hbm.at[pl.ds(nb, KW)], dv[p1])
                for cpy in rows(w + 1, p1):
                    cpy.start()
                for cpy in rows(w, p):
                    cpy.wait()

                def ed(k, c2):
                    x = elr[p][k, :] + err[p][k, :]
                    x = jnp.maximum(x, x * 0.2)
                    ee_v[k, :] = jnp.exp(x)
                    return c2

                lax.fori_loop(0, KW, ed, 0)
                base = tb + w * KW
                pltpu.sync_copy(ee_v, ee_hbm.at[pl.ds(base, KW)])
                pltpu.sync_copy(ee_v, den_s.at[dv[p]], add=True)
            return carry

        lax.fori_loop(0, nwin // 2, pair, 0)
        for cpy in rows(0, 0):
            cpy.wait()
        plsc.subcore_barrier()
        pltpu.sync_copy(den_s.at[pl.ds(sid * rpt, rpt)],
                        den_hbm.at[cid, pl.ds(sid * rpt, rpt)])

    return pl.kernel(
        body,
        out_type=[jax.ShapeDtypeStruct((epad, 16), F32),
                  jax.ShapeDtypeStruct((2, npad, 16), F32)],
        mesh=_mesh(),
        compiler_params=pltpu.CompilerParams(use_tc_tiling_on_sc=False, needs_layout_passes=False),
        scratch_types=[pltpu.VMEM((KW,), jnp.int32)] * 4
        + [pltpu.VMEM((KW, 16), F32)] * 5
        + [pltpu.VMEM_SHARED((npad, 16), F32)]
        + [pltpu.SemaphoreType.DMA] * 2,
    )


def _make_pass_b(n, npad, epad):
    ept = epad // NW
    nwin = ept // KW
    rpt = npad // 16

    def body(h0_hbm, h1_hbm, h2_hbm, h3_hbm, ee_hbm, src_hbm, dst_hbm,
             z32_hbm,
             r0_hbm, r1_hbm, r2_hbm, r3_hbm,
             s0, s1, d0, d1, ee0, ee1, hv0, hv1, m_v, acc_s,
             rs0, rs1):
        cid = lax.axis_index("c")
        sid = lax.axis_index("s")
        wid = sid * 2 + cid
        tb = wid * ept
        iota = lax.iota(jnp.int32, 16)
        hc_hbm = (h0_hbm, h1_hbm, h2_hbm, h3_hbm)
        raw_hbm = (r0_hbm, r1_hbm, r2_hbm, r3_hbm)
        sv, dv = (s0, s1), (d0, d1)
        eev, hv, rsem = (ee0, ee1), (hv0, hv1), (rs0, rs1)

        for c in range(4):
            pat0 = 4 * c + lax.shift_right_logical(iota, 3)
            pat1 = 4 * c + 2 + lax.shift_right_logical(iota, 3)

            def rows(w, p, _c=c):
                return (pltpu.make_async_copy(
                            ee_hbm.at[pl.ds(tb + lax.rem(w, nwin) * KW, KW)],
                            eev[p], rsem[p]),
                        pltpu.make_async_copy(hc_hbm[_c].at[sv[p]], hv[p],
                                              rsem[p]))

            pltpu.sync_copy(z32_hbm, acc_s.at[pl.ds(sid * rpt, rpt)])
            plsc.subcore_barrier()

            pltpu.sync_copy(src_hbm.at[pl.ds(tb, KW)], s0)
            pltpu.sync_copy(dst_hbm.at[pl.ds(tb, KW)], d0)
            for cpy in rows(0, 0):
                cpy.start()

            def pair(j, carry, _p0=pat0, _p1=pat1, _rows=rows):
                for p in range(2):
                    w = 2 * j + p
                    p1 = (p + 1) % 2
                    nb = tb + lax.rem(w + 1, nwin) * KW
                    pltpu.sync_copy(src_hbm.at[pl.ds(nb, KW)], sv[p1])
                    pltpu.sync_copy(dst_hbm.at[pl.ds(nb, KW)], dv[p1])
                    for cpy in _rows(w + 1, p1):
                        cpy.start()
                    for cpy in _rows(w, p):
                        cpy.wait()

                    def ed(k, c2):
                        rws = jnp.full((16,), k, jnp.int32)
                        e0 = plsc.load_gather(eev[p], [rws, _p0])
                        e1 = plsc.load_gather(eev[p], [rws, _p1])
                        m_v[k, pl.ds(0, 16)] = hv[p][k, pl.ds(0, 16)] * e0
                        m_v[k, pl.ds(16, 16)] = hv[p][k, pl.ds(16, 16)] * e1
                        return c2

                    lax.fori_loop(0, KW, ed, 0)
                    pltpu.sync_copy(m_v, acc_s.at[dv[p]], add=True)
                return carry

            lax.fori_loop(0, nwin // 2, pair, 0)
            for cpy in rows(0, 0):
                cpy.wait()
            plsc.subcore_barrier()
            pltpu.sync_copy(acc_s.at[pl.ds(sid * rpt, rpt)],
                            raw_hbm[c].at[cid, pl.ds(sid * rpt, rpt)])
            plsc.subcore_barrier()

    out = jax.ShapeDtypeStruct((2, npad, 32), F32)
    return pl.kernel(
        body,
        out_type=[out, out, out, out],
        mesh=_mesh(),
        compiler_params=pltpu.CompilerParams(use_tc_tiling_on_sc=False, needs_layout_passes=False),
        scratch_types=[pltpu.VMEM((KW,), jnp.int32)] * 4
        + [pltpu.VMEM((KW, 16), F32)] * 2
        + [pltpu.VMEM((KW, 32), F32)] * 3
        + [pltpu.VMEM_SHARED((npad, 32), F32)]
        + [pltpu.SemaphoreType.DMA] * 2,
    )


def _mlp2(x, w1, b1, w2, b2, relu_mid=True):
    n = x.shape[0]
    dout = w2.shape[1]

    def body(x_ref, w1_ref, b1_ref, w2_ref, b2_ref, o_ref):
        h = jnp.dot(x_ref[...], w1_ref[...], preferred_element_type=F32)
        h = jnp.maximum(h + b1_ref[...], 0.0)
        o_ref[...] = jnp.dot(h, w2_ref[...], preferred_element_type=F32) + b2_ref[...]

    return pl.pallas_call(
        body,
        grid=(n // BN,),
        in_specs=[pl.BlockSpec((BN, 128), lambda i: (i, 0)),
                  pl.BlockSpec((128, 128), lambda i: (0, 0)),
                  pl.BlockSpec((1, 128), lambda i: (0, 0)),
                  pl.BlockSpec((128, dout), lambda i: (0, 0)),
                  pl.BlockSpec((1, dout), lambda i: (0, 0))],
        out_specs=pl.BlockSpec((BN, dout), lambda i: (i, 0)),
        out_shape=jax.ShapeDtypeStruct((n, dout), F32),
    )(x, w1, b1.reshape(1, -1), w2, b2.reshape(1, -1))


def _project(h1, w, aal, aar):
    n = h1.shape[0]

    def body(h1_ref, w_ref, al_ref, ar_ref, *outs):
        h1b = h1_ref[...]
        for r in range(2):
            h = jnp.dot(h1b, w_ref[r], preferred_element_type=F32)
            for c in range(4):
                outs[r * 4 + c][...] = h[:, c * 32:(c + 1) * 32]
            outs[8 + 2 * r][...] = jnp.dot(h, al_ref[r], preferred_element_type=F32)
            outs[8 + 2 * r + 1][...] = jnp.dot(h, ar_ref[r], preferred_element_type=F32)

    chunk = jax.ShapeDtypeStruct((n, 32), F32)
    att = jax.ShapeDtypeStruct((n, 16), F32)
    return pl.pallas_call(
        body,
        grid=(n // BN,),
        in_specs=[pl.BlockSpec((BN, 128), lambda i: (i, 0)),
                  pl.BlockSpec((2, 128, 128), lambda i: (0, 0, 0)),
                  pl.BlockSpec((2, 128, 16), lambda i: (0, 0, 0)),
                  pl.BlockSpec((2, 128, 16), lambda i: (0, 0, 0))],
        out_specs=[pl.BlockSpec((BN, 32), lambda i: (i, 0))] * 8
        + [pl.BlockSpec((BN, 16), lambda i: (i, 0))] * 4,
        out_shape=[chunk] * 8 + [att] * 4,
    )(h1, w, aal, aar)


def _combine(raws0, raws1, den0, den1, h1, b, npad):
    n = h1.shape[0]

    def body(r00, r01, r02, r03, r10, r11, r12, r13, d0_ref, d1_ref,
             h1_ref, b_ref, o_ref):
        raws = ((r00, r01, r02, r03), (r10, r11, r12, r13))
        dens = (d0_ref, d1_ref)
        reps = []
        for r in range(2):
            d = dens[r][0] + dens[r][1]
            d = jnp.where(d == 0.0, 1.0, d)
            reps.append(jnp.repeat(1.0 / d, 8, axis=1))
        cols = []
        for c in range(4):
            col = jnp.zeros((BN, 32), F32)
            for r in range(2):
                s = raws[r][c][0] + raws[r][c][1]
                col = col + s * reps[r][:, c * 32:(c + 1) * 32] \
                    + b_ref[r, c * 32:(c + 1) * 32]
            cols.append(col)
        acc = jnp.concatenate(cols, axis=1)
        o_ref[...] = jnp.where(acc > 0, acc, 0.01 * acc) + h1_ref[...]

    rspec = pl.BlockSpec((2, BN, 32), lambda i: (0, i, 0))
    dspec = pl.BlockSpec((2, BN, 16), lambda i: (0, i, 0))
    return pl.pallas_call(
        body,
        grid=(n // BN,),
        in_specs=[rspec] * 8 + [dspec] * 2
        + [pl.BlockSpec((BN, 128), lambda i: (i, 0)),
           pl.BlockSpec((2, 128), lambda i: (0, 0))],
        out_specs=pl.BlockSpec((BN, 128), lambda i: (i, 0)),
        out_shape=jax.ShapeDtypeStruct((n, 128), F32),
    )(*raws0, *raws1, den0, den1, h1, b)


def kernel(x, edge_index_rel0, edge_index_rel1,
           embed_w1, embed_b1, embed_w2, embed_b2,
           gat_w, gat_al, gat_ar, gat_b,
           dec_w1, dec_b1, dec_w2, dec_b2):
    n = x.shape[0]
    e = edge_index_rel0.shape[1]
    L, R, heads, hd = gat_al.shape
    npad = ((n + 64 + 127) // 128) * 128
    quant = NW * KW * 4
    epad = ((e + quant - 1) // quant) * quant
    pad = epad - e
    rpt = npad // 16

    padsrc = jnp.zeros((pad,), jnp.int32)
    paddst = n + (jnp.arange(pad, dtype=jnp.int32) % 64)
    edges = []
    for ei in (edge_index_rel0, edge_index_rel1):
        edges.append((jnp.concatenate([ei[0], padsrc]),
                      jnp.concatenate([ei[1], paddst])))

    eye = (jnp.arange(128)[:, None] // hd == jnp.arange(heads)[None, :])
    aal = jnp.where(eye[None, None], gat_al.reshape(L, R, 128)[..., None], 0.0)
    aar = jnp.where(eye[None, None], gat_ar.reshape(L, R, 128)[..., None], 0.0)

    z16 = jnp.zeros((rpt, 16), F32)
    z32 = jnp.zeros((rpt, 32), F32)
    zpad16 = jnp.zeros((npad - n, 16), F32)

    pass_a = _make_pass_a(n, npad, epad)
    pass_b = _make_pass_b(n, npad, epad)

    h1 = _mlp2(x, embed_w1, embed_b1, embed_w2, embed_b2)
    for l in range(L):
        proj = _project(h1, gat_w[l], aal[l], aar[l])
        hcs = (proj[0:4], proj[4:8])
        els = (proj[8], proj[10])
        ers = (proj[9], proj[11])
        raws, dens = [], []
        for r in range(R):
            erp = jnp.concatenate([ers[r], zpad16], axis=0)
            ee, den = pass_a(els[r], erp, edges[r][0], edges[r][1], z16)
            raw = pass_b(*hcs[r], ee, edges[r][0], edges[r][1], z32)
            raws.append(raw)
            dens.append(den)
        h1 = _combine(raws[0], raws[1], dens[0], dens[1], h1, gat_b[l], npad)
    return _mlp2(h1, dec_w1, dec_b1, dec_w2, dec_b2)


# speedup vs baseline: 37.6131x; 1.0747x over previous
import functools

import jax
import jax.numpy as jnp
from jax import lax
from jax.experimental import pallas as pl
from jax.experimental.pallas import tpu as pltpu
from jax.experimental.pallas import tpu_sc as plsc

F32 = jnp.float32
NW = 32
KW = 128
BN = 1000


def _mesh():
    return plsc.VectorSubcoreMesh(core_axis_name="c", subcore_axis_name="s")


def _make_pass_a(n, npad, epad):
    ept = epad // NW
    nwin = ept // KW
    rpt = npad // 16

    def body(el_hbm, er_hbm, src_hbm, dst_hbm, z16_hbm,
             ee_hbm, den_hbm,
             s0, s1, s2, s3, d0, d1, d2, d3,
             elr0, elr1, err0, err1, ee_v, den_s,
             i0, i1, i2, i3, rs0, rs1):
        cid = lax.axis_index("c")
        sid = lax.axis_index("s")
        wid = sid * 2 + cid
        tb = wid * ept
        sv, dv = (s0, s1, s2, s3), (d0, d1, d2, d3)
        elr, err, rsem = (elr0, elr1), (err0, err1), (rs0, rs1)
        isem = (i0, i1, i2, i3)
        pltpu.sync_copy(z16_hbm, den_s.at[pl.ds(sid * rpt, rpt)])
        plsc.subcore_barrier()

        def idx(w, b):
            nb = tb + lax.rem(w, nwin) * KW
            return (pltpu.make_async_copy(src_hbm.at[pl.ds(nb, KW)], sv[b],
                                          isem[b]),
                    pltpu.make_async_copy(dst_hbm.at[pl.ds(nb, KW)], dv[b],
                                          isem[b]))

        def rows(w, b, p):
            return (pltpu.make_async_copy(el_hbm.at[sv[b]], elr[p], rsem[p]),
                    pltpu.make_async_copy(er_hbm.at[dv[b]], err[p], rsem[p]))

        for cpy in idx(0, 0) + idx(1, 1):
            cpy.start()
        for cpy in idx(0, 0):
            cpy.wait()
        for cpy in rows(0, 0, 0):
            cpy.start()

        def pair(j, carry):
            for b in range(4):
                w = 4 * j + b
                p = b % 2
                b1, b2, p1 = (b + 1) % 4, (b + 2) % 4, (b + 1) % 2
                for cpy in idx(w + 2, b2):
                    cpy.start()
                for cpy in idx(w + 1, b1):
                    cpy.wait()
                for cpy in rows(w + 1, b1, p1):
                    cpy.start()
                for cpy in rows(w, b, p):
                    cpy.wait()

                def ed(k, c2):
                    x = elr[p][k, :] + err[p][k, :]
                    x = jnp.maximum(x, x * 0.2)
                    ee_v[k, :] = jnp.exp(x)
                    return c2

                lax.fori_loop(0, KW, ed, 0)
                base = tb + w * KW
                pltpu.sync_copy(ee_v, ee_hbm.at[pl.ds(base, KW)])
                pltpu.sync_copy(ee_v, den_s.at[dv[b]], add=True)
            return carry

        lax.fori_loop(0, nwin // 4, pair, 0)
        for cpy in rows(nwin, 0, 0):
            cpy.wait()
        for cpy in idx(nwin + 1, 1):
            cpy.wait()
        plsc.subcore_barrier()
        pltpu.sync_copy(den_s.at[pl.ds(sid * rpt, rpt)],
                        den_hbm.at[cid, pl.ds(sid * rpt, rpt)])

    return pl.kernel(
        body,
        out_type=[jax.ShapeDtypeStruct((epad, 16), F32),
                  jax.ShapeDtypeStruct((2, npad, 16), F32)],
        mesh=_mesh(),
        compiler_params=pltpu.CompilerParams(use_tc_tiling_on_sc=False, needs_layout_passes=False),
        scratch_types=[pltpu.VMEM((KW,), jnp.int32)] * 8
        + [pltpu.VMEM((KW, 16), F32)] * 5
        + [pltpu.VMEM_SHARED((npad, 16), F32)]
        + [pltpu.SemaphoreType.DMA] * 6,
    )


def _make_pass_b(n, npad, epad):
    ept = epad // NW
    nwin = ept // KW
    rpt = npad // 16

    def body(h0_hbm, h1_hbm, h2_hbm, h3_hbm, ee_hbm, src_hbm, dst_hbm,
             z32_hbm,
             r0_hbm, r1_hbm, r2_hbm, r3_hbm,
             s0, s1, s2, s3, d0, d1, d2, d3,
             ee0, ee1, hv0, hv1, m_v, acc_s,
             i0, i1, i2, i3, rs0, rs1):
        cid = lax.axis_index("c")
        sid = lax.axis_index("s")
        wid = sid * 2 + cid
        tb = wid * ept
        iota = lax.iota(jnp.int32, 16)
        hc_hbm = (h0_hbm, h1_hbm, h2_hbm, h3_hbm)
        raw_hbm = (r0_hbm, r1_hbm, r2_hbm, r3_hbm)
        sv, dv = (s0, s1, s2, s3), (d0, d1, d2, d3)
        eev, hv, rsem = (ee0, ee1), (hv0, hv1), (rs0, rs1)
        isem = (i0, i1, i2, i3)

        def idx(w, b):
            nb = tb + lax.rem(w, nwin) * KW
            return (pltpu.make_async_copy(src_hbm.at[pl.ds(nb, KW)], sv[b],
                                          isem[b]),
                    pltpu.make_async_copy(dst_hbm.at[pl.ds(nb, KW)], dv[b],
                                          isem[b]))

        for c in range(4):
            pat0 = 4 * c + lax.shift_right_logical(iota, 3)
            pat1 = 4 * c + 2 + lax.shift_right_logical(iota, 3)

            def rows(w, b, p, _c=c):
                return (pltpu.make_async_copy(
                            ee_hbm.at[pl.ds(tb + lax.rem(w, nwin) * KW, KW)],
                            eev[p], rsem[p]),
                        pltpu.make_async_copy(hc_hbm[_c].at[sv[b]], hv[p],
                                              rsem[p]))

            pltpu.sync_copy(z32_hbm, acc_s.at[pl.ds(sid * rpt, rpt)])
            plsc.subcore_barrier()

            for cpy in idx(0, 0) + idx(1, 1):
                cpy.start()
            for cpy in idx(0, 0):
                cpy.wait()
            for cpy in rows(0, 0, 0):
                cpy.start()

            def pair(j, carry, _p0=pat0, _p1=pat1, _rows=rows):
                for b in range(4):
                    w = 4 * j + b
                    p = b % 2
                    b1, b2, p1 = (b + 1) % 4, (b + 2) % 4, (b + 1) % 2
                    for cpy in idx(w + 2, b2):
                        cpy.start()
                    for cpy in idx(w + 1, b1):
                        cpy.wait()
                    for cpy in _rows(w + 1, b1, p1):
                        cpy.start()
                    for cpy in _rows(w, b, p):
                        cpy.wait()

                    def ed(k, c2):
                        rws = jnp.full((16,), k, jnp.int32)
                        e0 = plsc.load_gather(eev[p], [rws, _p0])
                        e1 = plsc.load_gather(eev[p], [rws, _p1])
                        m_v[k, pl.ds(0, 16)] = hv[p][k, pl.ds(0, 16)] * e0
                        m_v[k, pl.ds(16, 16)] = hv[p][k, pl.ds(16, 16)] * e1
                        return c2

                    lax.fori_loop(0, KW, ed, 0)
                    pltpu.sync_copy(m_v, acc_s.at[dv[b]], add=True)
                return carry

            lax.fori_loop(0, nwin // 4, pair, 0)
            for cpy in rows(nwin, 0, 0):
                cpy.wait()
            for cpy in idx(nwin + 1, 1):
                cpy.wait()
            plsc.subcore_barrier()
            pltpu.sync_copy(acc_s.at[pl.ds(sid * rpt, rpt)],
                            raw_hbm[c].at[cid, pl.ds(sid * rpt, rpt)])
            plsc.subcore_barrier()

    out = jax.ShapeDtypeStruct((2, npad, 32), F32)
    return pl.kernel(
        body,
        out_type=[out, out, out, out],
        mesh=_mesh(),
        compiler_params=pltpu.CompilerParams(use_tc_tiling_on_sc=False, needs_layout_passes=False),
        scratch_types=[pltpu.VMEM((KW,), jnp.int32)] * 8
        + [pltpu.VMEM((KW, 16), F32)] * 2
        + [pltpu.VMEM((KW, 32), F32)] * 3
        + [pltpu.VMEM_SHARED((npad, 32), F32)]
        + [pltpu.SemaphoreType.DMA] * 6,
    )


def _mlp2(x, w1, b1, w2, b2, relu_mid=True):
    n = x.shape[0]
    dout = w2.shape[1]

    def body(x_ref, w1_ref, b1_ref, w2_ref, b2_ref, o_ref):
        h = jnp.dot(x_ref[...], w1_ref[...], preferred_element_type=F32)
        h = jnp.maximum(h + b1_ref[...], 0.0)
        o_ref[...] = jnp.dot(h, w2_ref[...], preferred_element_type=F32) + b2_ref[...]

    return pl.pallas_call(
        body,
        grid=(n // BN,),
        in_specs=[pl.BlockSpec((BN, 128), lambda i: (i, 0)),
                  pl.BlockSpec((128, 128), lambda i: (0, 0)),
                  pl.BlockSpec((1, 128), lambda i: (0, 0)),
                  pl.BlockSpec((128, dout), lambda i: (0, 0)),
                  pl.BlockSpec((1, dout), lambda i: (0, 0))],
        out_specs=pl.BlockSpec((BN, dout), lambda i: (i, 0)),
        out_shape=jax.ShapeDtypeStruct((n, dout), F32),
    )(x, w1, b1.reshape(1, -1), w2, b2.reshape(1, -1))


def _project(h1, w, aal, aar):
    n = h1.shape[0]

    def body(h1_ref, w_ref, al_ref, ar_ref, *outs):
        h1b = h1_ref[...]
        for r in range(2):
            h = jnp.dot(h1b, w_ref[r], preferred_element_type=F32)
            for c in range(4):
                outs[r * 4 + c][...] = h[:, c * 32:(c + 1) * 32]
            outs[8 + 2 * r][...] = jnp.dot(h, al_ref[r], preferred_element_type=F32)
            outs[8 + 2 * r + 1][...] = jnp.dot(h, ar_ref[r], preferred_element_type=F32)

    chunk = jax.ShapeDtypeStruct((n, 32), F32)
    att = jax.ShapeDtypeStruct((n, 16), F32)
    return pl.pallas_call(
        body,
        grid=(n // BN,),
        in_specs=[pl.BlockSpec((BN, 128), lambda i: (i, 0)),
                  pl.BlockSpec((2, 128, 128), lambda i: (0, 0, 0)),
                  pl.BlockSpec((2, 128, 16), lambda i: (0, 0, 0)),
                  pl.BlockSpec((2, 128, 16), lambda i: (0, 0, 0))],
        out_specs=[pl.BlockSpec((BN, 32), lambda i: (i, 0))] * 8
        + [pl.BlockSpec((BN, 16), lambda i: (i, 0))] * 4,
        out_shape=[chunk] * 8 + [att] * 4,
    )(h1, w, aal, aar)


def _combine(raws0, raws1, den0, den1, h1, b, npad):
    n = h1.shape[0]

    def body(r00, r01, r02, r03, r10, r11, r12, r13, d0_ref, d1_ref,
             h1_ref, b_ref, o_ref):
        raws = ((r00, r01, r02, r03), (r10, r11, r12, r13))
        dens = (d0_ref, d1_ref)
        reps = []
        for r in range(2):
            d = dens[r][0] + dens[r][1]
            d = jnp.where(d == 0.0, 1.0, d)
            reps.append(jnp.repeat(1.0 / d, 8, axis=1))
        cols = []
        for c in range(4):
            col = jnp.zeros((BN, 32), F32)
            for r in range(2):
                s = raws[r][c][0] + raws[r][c][1]
                col = col + s * reps[r][:, c * 32:(c + 1) * 32] \
                    + b_ref[r, c * 32:(c + 1) * 32]
            cols.append(col)
        acc = jnp.concatenate(cols, axis=1)
        o_ref[...] = jnp.where(acc > 0, acc, 0.01 * acc) + h1_ref[...]

    rspec = pl.BlockSpec((2, BN, 32), lambda i: (0, i, 0))
    dspec = pl.BlockSpec((2, BN, 16), lambda i: (0, i, 0))
    return pl.pallas_call(
        body,
        grid=(n // BN,),
        in_specs=[rspec] * 8 + [dspec] * 2
        + [pl.BlockSpec((BN, 128), lambda i: (i, 0)),
           pl.BlockSpec((2, 128), lambda i: (0, 0))],
        out_specs=pl.BlockSpec((BN, 128), lambda i: (i, 0)),
        out_shape=jax.ShapeDtypeStruct((n, 128), F32),
    )(*raws0, *raws1, den0, den1, h1, b)


def kernel(x, edge_index_rel0, edge_index_rel1,
           embed_w1, embed_b1, embed_w2, embed_b2,
           gat_w, gat_al, gat_ar, gat_b,
           dec_w1, dec_b1, dec_w2, dec_b2):
    n = x.shape[0]
    e = edge_index_rel0.shape[1]
    L, R, heads, hd = gat_al.shape
    npad = ((n + 64 + 127) // 128) * 128
    quant = NW * KW * 4
    epad = ((e + quant - 1) // quant) * quant
    pad = epad - e
    rpt = npad // 16

    padsrc = jnp.zeros((pad,), jnp.int32)
    paddst = n + (jnp.arange(pad, dtype=jnp.int32) % 64)
    edges = []
    for ei in (edge_index_rel0, edge_index_rel1):
        edges.append((jnp.concatenate([ei[0], padsrc]),
                      jnp.concatenate([ei[1], paddst])))

    eye = (jnp.arange(128)[:, None] // hd == jnp.arange(heads)[None, :])
    aal = jnp.where(eye[None, None], gat_al.reshape(L, R, 128)[..., None], 0.0)
    aar = jnp.where(eye[None, None], gat_ar.reshape(L, R, 128)[..., None], 0.0)

    z16 = jnp.zeros((rpt, 16), F32)
    z32 = jnp.zeros((rpt, 32), F32)
    zpad16 = jnp.zeros((npad - n, 16), F32)

    pass_a = _make_pass_a(n, npad, epad)
    pass_b = _make_pass_b(n, npad, epad)

    h1 = _mlp2(x, embed_w1, embed_b1, embed_w2, embed_b2)
    for l in range(L):
        proj = _project(h1, gat_w[l], aal[l], aar[l])
        hcs = (proj[0:4], proj[4:8])
        els = (proj[8], proj[10])
        ers = (proj[9], proj[11])
        raws, dens = [], []
        for r in range(R):
            erp = jnp.concatenate([ers[r], zpad16], axis=0)
            ee, den = pass_a(els[r], erp, edges[r][0], edges[r][1], z16)
            raw = pass_b(*hcs[r], ee, edges[r][0], edges[r][1], z32)
            raws.append(raw)
            dens.append(den)
        h1 = _combine(raws[0], raws[1], dens[0], dens[1], h1, gat_b[l], npad)
    return _mlp2(h1, dec_w1, dec_b1, dec_w2, dec_b2)
